# Initial kernel scaffold; baseline (speedup 1.0000x reference)
import jax
import jax.numpy as jnp
from jax.experimental import pallas as pl


def kernel(h, edge_index):
    raise NotImplementedError("write your pallas kernel here")


# speedup vs baseline: 2.6046x; 2.6046x over previous
import functools

import jax
import jax.numpy as jnp
from jax import lax
from jax.experimental import pallas as pl
from jax.experimental.pallas import tpu as pltpu
from jax.experimental.pallas import tpu_sc as plsc

D = 128
C = 80
L = 16


def _make_sc_kernel(E, NW):
    epw = E // NW
    nchunk = epw // C

    mesh = plsc.VectorSubcoreMesh(core_axis_name="c", subcore_axis_name="s")

    @functools.partial(
        pl.kernel,
        mesh=mesh,
        out_type=jax.ShapeDtypeStruct((E,), jnp.float32),
        scratch_types=[
            pltpu.VMEM((C,), jnp.int32),
            pltpu.VMEM((C,), jnp.int32),
            pltpu.VMEM((C, D), jnp.float32),
            pltpu.VMEM((C, D), jnp.float32),
            pltpu.VMEM((C,), jnp.float32),
            pltpu.SemaphoreType.DMA,
            pltpu.SemaphoreType.DMA,
        ],
        compiler_params=pltpu.CompilerParams(needs_layout_passes=False),
    )
    def sc_k(h_hbm, src_hbm, dst_hbm, out_hbm,
             idx_u, idx_v, rows_u, rows_v, out_v, sem_u, sem_v):
        cid = lax.axis_index("c")
        sid = lax.axis_index("s")
        wid = sid * 2 + cid
        lanes = lax.iota(jnp.int32, L)

        def chunk_body(ci, carry):
            base = wid * epw + ci * C
            pltpu.sync_copy(src_hbm.at[pl.ds(base, C)], idx_u)
            pltpu.sync_copy(dst_hbm.at[pl.ds(base, C)], idx_v)
            cp_u = pltpu.async_copy(h_hbm.at[idx_u], rows_u, sem_u)
            cp_v = pltpu.async_copy(h_hbm.at[idx_v], rows_v, sem_v)
            cp_u.wait()
            cp_v.wait()

            def group_body(g, carry2):
                vec = jnp.zeros((L,), jnp.float32)
                for j in range(L):
                    e = g * L + j
                    acc = rows_u[e, pl.ds(0, L)] * rows_v[e, pl.ds(0, L)]
                    for k in range(1, D // L):
                        acc = acc + (rows_u[e, pl.ds(k * L, L)]
                                     * rows_v[e, pl.ds(k * L, L)])
                    s = jnp.sum(acc)
                    vec = jnp.where(lanes == j, s, vec)
                out_v[pl.ds(g * L, L)] = vec
                return carry2

            lax.fori_loop(0, C // L, group_body, 0)
            pltpu.sync_copy(out_v, out_hbm.at[pl.ds(base, C)])
            return carry

        lax.fori_loop(0, nchunk, chunk_body, 0)

    return sc_k


def kernel(h, edge_index):
    E = edge_index.shape[1]
    info = plsc.get_sparse_core_info()
    NW = info.num_cores * info.num_subcores
    src = edge_index[0].astype(jnp.int32)
    dst = edge_index[1].astype(jnp.int32)
    score = _make_sc_kernel(E, NW)(h, src, dst)
    return score[:, None]


# speedup vs baseline: 3.1678x; 1.2162x over previous
import functools

import jax
import jax.numpy as jnp
from jax import lax
from jax.experimental import pallas as pl
from jax.experimental.pallas import tpu as pltpu
from jax.experimental.pallas import tpu_sc as plsc

D = 128
C = 80
L = 16
NBUF = 4


def _make_sc_kernel(E, NW):
    epw = E // NW
    nchunk = epw // C
    niter = nchunk // NBUF
    ntail = nchunk - niter * NBUF

    mesh = plsc.VectorSubcoreMesh(core_axis_name="c", subcore_axis_name="s")

    @functools.partial(
        pl.kernel,
        mesh=mesh,
        out_type=jax.ShapeDtypeStruct((E,), jnp.float32),
        scratch_types=[
            pltpu.VMEM((epw,), jnp.int32),
            pltpu.VMEM((epw,), jnp.int32),
            pltpu.VMEM((epw,), jnp.float32),
        ] + [pltpu.VMEM((C, D), jnp.float32)] * (2 * NBUF)
          + [pltpu.SemaphoreType.DMA] * (2 * NBUF),
        compiler_params=pltpu.CompilerParams(needs_layout_passes=False),
    )
    def sc_k(h_hbm, src_hbm, dst_hbm, out_hbm,
             idx_u_all, idx_v_all, out_all, *bufs_sems):
        rows_u = bufs_sems[0:NBUF]
        rows_v = bufs_sems[NBUF:2 * NBUF]
        sems_u = bufs_sems[2 * NBUF:3 * NBUF]
        sems_v = bufs_sems[3 * NBUF:4 * NBUF]

        cid = lax.axis_index("c")
        sid = lax.axis_index("s")
        wid = sid * 2 + cid
        base = wid * epw
        lanes = lax.iota(jnp.int32, L)

        pltpu.sync_copy(src_hbm.at[pl.ds(base, epw)], idx_u_all)
        pltpu.sync_copy(dst_hbm.at[pl.ds(base, epw)], idx_v_all)

        def issue(c, j):
            cu = pltpu.async_copy(
                h_hbm.at[idx_u_all.at[pl.ds(c * C, C)]], rows_u[j], sems_u[j])
            cv = pltpu.async_copy(
                h_hbm.at[idx_v_all.at[pl.ds(c * C, C)]], rows_v[j], sems_v[j])
            return cu, cv

        def compute(j, cbase):
            ru = rows_u[j]
            rv = rows_v[j]

            def group_body(g, carry):
                vec = jnp.zeros((L,), jnp.float32)
                for jj in range(L):
                    e = g * L + jj
                    acc = ru[e, pl.ds(0, L)] * rv[e, pl.ds(0, L)]
                    for k in range(1, D // L):
                        acc = acc + (ru[e, pl.ds(k * L, L)]
                                     * rv[e, pl.ds(k * L, L)])
                    s = jnp.sum(acc)
                    vec = jnp.where(lanes == jj, s, vec)
                out_all[pl.ds(cbase + g * L, L)] = vec
                return carry

            lax.fori_loop(0, C // L, group_body, 0)

        def round_body(i, carry):
            c0 = i * NBUF
            cps = [issue(c0 + j, j) for j in range(NBUF)]
            for j in range(NBUF):
                cu, cv = cps[j]
                cu.wait()
                cv.wait()
                compute(j, (c0 + j) * C)
            return carry

        lax.fori_loop(0, niter, round_body, 0)

        tail0 = niter * NBUF
        cps = [issue(tail0 + j, j) for j in range(ntail)]
        for j in range(ntail):
            cu, cv = cps[j]
            cu.wait()
            cv.wait()
            compute(j, (tail0 + j) * C)

        pltpu.sync_copy(out_all, out_hbm.at[pl.ds(base, epw)])

    return sc_k


def kernel(h, edge_index):
    E = edge_index.shape[1]
    info = plsc.get_sparse_core_info()
    NW = info.num_cores * info.num_subcores
    src = edge_index[0].astype(jnp.int32)
    dst = edge_index[1].astype(jnp.int32)
    score = _make_sc_kernel(E, NW)(h, src, dst)
    return score[:, None]
